# Initial kernel scaffold; baseline (speedup 1.0000x reference)
import jax
import jax.numpy as jnp
from jax.experimental import pallas as pl


def kernel(x, edge_index, edge_attr, batch, x_emb1, x_emb2, edge_tab1, edge_tab2, mlp_w1, mlp_b1, mlp_w2, mlp_b2, bn_gamma, bn_beta, feat_w, feat_b, out_w1, out_b1, out_w2, out_b2):
    raise NotImplementedError("write your pallas kernel here")


# speedup vs baseline: 6.0630x; 6.0630x over previous
import functools

import jax
import jax.numpy as jnp
from jax import lax
from jax.experimental import pallas as pl
from jax.experimental.pallas import tpu as pltpu
from jax.experimental.pallas import tpu_sc as plsc

NC = 2
NS = 16
NW = NC * NS
CH = 128
NGRP = 256


def _make_spmm(n, n_pad, ep, d):
    per_tile = ep // NW
    chunks = per_tile // CH
    rows_per_tile = n_pad // NS
    mesh = plsc.VectorSubcoreMesh(core_axis_name="c", subcore_axis_name="s")

    @functools.partial(
        pl.kernel,
        out_type=jax.ShapeDtypeStruct((NC, n_pad, d), jnp.float32),
        mesh=mesh,
        scratch_types=[
            pltpu.VMEM((CH,), jnp.int32),
            pltpu.VMEM((CH,), jnp.int32),
            pltpu.VMEM((CH, d), jnp.float32),
            pltpu.VMEM_SHARED((n_pad, d), jnp.float32),
            pltpu.SemaphoreType.DMA,
        ],
    )
    def spmm(h_hbm, src_hbm, dst_hbm, zeros_hbm, out_hbm,
             idx_s, idx_d, rows_v, acc_sh, sem):
        c = lax.axis_index("c")
        s = lax.axis_index("s")
        wid = s * NC + c
        row0 = s * rows_per_tile
        pltpu.sync_copy(zeros_hbm, acc_sh.at[pl.ds(row0, rows_per_tile)])
        plsc.subcore_barrier()
        e0 = wid * per_tile

        def body(i, carry):
            base = e0 + i * CH
            pltpu.sync_copy(src_hbm.at[pl.ds(base, CH)], idx_s)
            pltpu.sync_copy(dst_hbm.at[pl.ds(base, CH)], idx_d)
            pltpu.async_copy(h_hbm.at[idx_s], rows_v, sem).wait()
            pltpu.sync_copy(rows_v, acc_sh.at[idx_d], add=True)
            return carry

        lax.fori_loop(0, chunks, body, 0)
        plsc.subcore_barrier()
        pltpu.sync_copy(acc_sh.at[pl.ds(row0, rows_per_tile)],
                        out_hbm.at[c, pl.ds(row0, rows_per_tile)])

    return spmm


def _make_counts(n_pad, ep):
    per_tile = ep // NW
    chunks = per_tile // CH
    rows_per_tile = n_pad // NS
    K = 16
    mesh = plsc.VectorSubcoreMesh(core_axis_name="c", subcore_axis_name="s")

    @functools.partial(
        pl.kernel,
        out_type=jax.ShapeDtypeStruct((NC, n_pad * K), jnp.float32),
        mesh=mesh,
        scratch_types=[
            pltpu.VMEM((CH,), jnp.int32),
            pltpu.VMEM((CH,), jnp.int32),
            pltpu.VMEM((CH,), jnp.int32),
            pltpu.VMEM((CH,), jnp.float32),
            pltpu.VMEM_SHARED((n_pad * K,), jnp.float32),
        ],
    )
    def counts_k(dst_hbm, ea0_hbm, ea1_hbm, zeros_hbm, out_hbm,
                 idx_d, ea0_v, ea1_v, ones_v, cnt_sh):
        c = lax.axis_index("c")
        s = lax.axis_index("s")
        wid = s * NC + c
        row0 = s * rows_per_tile * K
        pltpu.sync_copy(zeros_hbm, cnt_sh.at[pl.ds(row0, rows_per_tile * K)])
        for g in range(CH // 16):
            ones_v[pl.ds(g * 16, 16)] = jnp.ones((16,), jnp.float32)
        plsc.subcore_barrier()
        e0 = wid * per_tile

        def body(i, carry):
            base = e0 + i * CH
            pltpu.sync_copy(dst_hbm.at[pl.ds(base, CH)], idx_d)
            pltpu.sync_copy(ea0_hbm.at[pl.ds(base, CH)], ea0_v)
            pltpu.sync_copy(ea1_hbm.at[pl.ds(base, CH)], ea1_v)
            for g in range(CH // 16):
                sl = pl.ds(g * 16, 16)
                idx_d[sl] = idx_d[sl] * K + ea0_v[sl] * 3 + ea1_v[sl]
            pltpu.sync_copy(ones_v, cnt_sh.at[idx_d], add=True)
            return carry

        lax.fori_loop(0, chunks, body, 0)
        plsc.subcore_barrier()
        pltpu.sync_copy(cnt_sh.at[pl.ds(row0, rows_per_tile * K)],
                        out_hbm.at[c, pl.ds(row0, rows_per_tile * K)])

    return counts_k


def _embed_body(x_ref, e1_ref, e2_ref, o_ref):
    x0 = x_ref[:, 0:1]
    x1 = x_ref[:, 1:2]
    oh1 = (x0 == lax.broadcasted_iota(jnp.int32, (1, 128), 1)).astype(jnp.float32)
    oh2 = (x1 == lax.broadcasted_iota(jnp.int32, (1, 8), 1)).astype(jnp.float32)
    e1 = e1_ref[...]
    e1p = jnp.concatenate([e1, jnp.zeros((128 - e1.shape[0], e1.shape[1]),
                                         jnp.float32)], axis=0)
    e2 = e2_ref[...]
    e2p = jnp.concatenate([e2, jnp.zeros((8 - e2.shape[0], e2.shape[1]),
                                         jnp.float32)], axis=0)
    o_ref[...] = (jnp.dot(oh1, e1p, preferred_element_type=jnp.float32, precision=lax.Precision.HIGHEST)
                  + jnp.dot(oh2, e2p, preferred_element_type=jnp.float32, precision=lax.Precision.HIGHEST))


def _layer_body(relu_out, n, agg_ref, cnt_ref, h_ref, t1_ref, t2_ref,
                w1_ref, b1_ref, w2_ref, b2_ref, g_ref, bt_ref, o_ref):
    aggp = agg_ref[0][:n, :] + agg_ref[1][:n, :]
    cnt = cnt_ref[0][:n, :] + cnt_ref[1][:n, :]
    k = lax.broadcasted_iota(jnp.int32, (16, 1), 0)
    m1 = ((k // 3) == lax.broadcasted_iota(jnp.int32, (1, 5), 1)).astype(jnp.float32)
    m2 = (((k % 3) == lax.broadcasted_iota(jnp.int32, (1, 3), 1))
          & (k < 15)).astype(jnp.float32)
    t12 = (jnp.dot(m1, t1_ref[...], preferred_element_type=jnp.float32, precision=lax.Precision.HIGHEST)
           + jnp.dot(m2, t2_ref[...], preferred_element_type=jnp.float32, precision=lax.Precision.HIGHEST))
    h = h_ref[...]
    agg = (aggp + h + jnp.dot(cnt, t12, preferred_element_type=jnp.float32, precision=lax.Precision.HIGHEST)
           + t12[12:13, :])
    t = jnp.maximum(jnp.dot(agg, w1_ref[...],
                            preferred_element_type=jnp.float32) + b1_ref[...], 0.0)
    h2 = jnp.dot(t, w2_ref[...], preferred_element_type=jnp.float32) + b2_ref[...]
    mean = jnp.mean(h2, axis=0, keepdims=True)
    dz = h2 - mean
    var = jnp.mean(dz * dz, axis=0, keepdims=True)
    h2n = dz / jnp.sqrt(var + 1e-5) * g_ref[...] + bt_ref[...]
    if relu_out:
        h2n = jnp.maximum(h2n, 0.0)
    o_ref[...] = h2n


def _final_body(b_ref, h_ref, fw_ref, fb_ref, w1_ref, b1_ref, w2_ref, b2_ref,
                hf_ref, o_ref):
    ohbt = (lax.broadcasted_iota(jnp.int32, (NGRP, 1), 0)
            == b_ref[...]).astype(jnp.float32)
    sums = jnp.dot(ohbt, h_ref[...], preferred_element_type=jnp.float32, precision=lax.Precision.HIGHEST)
    cnts = jnp.sum(ohbt, axis=1, keepdims=True)
    pooled = sums / jnp.maximum(cnts, 1.0)
    hf = jnp.dot(pooled, fw_ref[...], preferred_element_type=jnp.float32) + fb_ref[...]
    r = jnp.maximum(jnp.dot(hf, w1_ref[...],
                            preferred_element_type=jnp.float32) + b1_ref[...], 0.0)
    hf_ref[...] = hf
    o_ref[...] = jnp.dot(r, w2_ref[...],
                         preferred_element_type=jnp.float32) + b2_ref[...]


def kernel(x, edge_index, edge_attr, batch, x_emb1, x_emb2, edge_tab1,
           edge_tab2, mlp_w1, mlp_b1, mlp_w2, mlp_b2, bn_gamma, bn_beta,
           feat_w, feat_b, out_w1, out_b1, out_w2, out_b2):
    n = x.shape[0]
    d = x_emb1.shape[1]
    e = edge_index.shape[1]
    num_layers = mlp_w1.shape[0]

    n_pad = ((n + 1 + NS * 8 - 1) // (NS * 8)) * (NS * 8)
    ep = ((e + NW * CH - 1) // (NW * CH)) * (NW * CH)
    pad = ep - e

    src_p = jnp.concatenate([edge_index[0],
                             jnp.zeros((pad,), jnp.int32)])
    dst_p = jnp.concatenate([edge_index[1],
                             jnp.full((pad,), n, jnp.int32)])
    ea0_p = jnp.concatenate([edge_attr[:, 0], jnp.zeros((pad,), jnp.int32)])
    ea1_p = jnp.concatenate([edge_attr[:, 1], jnp.zeros((pad,), jnp.int32)])
    zeros_d = jnp.zeros((n_pad // NS, d), jnp.float32)
    zeros_k = jnp.zeros((n_pad // NS * 16,), jnp.float32)

    embed = pl.pallas_call(
        _embed_body, out_shape=jax.ShapeDtypeStruct((n, d), jnp.float32))
    h = embed(x, x_emb1, x_emb2)

    counts_k = _make_counts(n_pad, ep)
    cnt = counts_k(dst_p, ea0_p, ea1_p, zeros_k).reshape(NC, n_pad, 16)

    spmm = _make_spmm(n, n_pad, ep, d)
    for l in range(num_layers):
        aggp = spmm(h, src_p, dst_p, zeros_d)
        layer = pl.pallas_call(
            functools.partial(_layer_body, l < num_layers - 1, n),
            out_shape=jax.ShapeDtypeStruct((n, d), jnp.float32))
        h = layer(aggp, cnt, h, edge_tab1[l], edge_tab2[l], mlp_w1[l],
                  mlp_b1[l][None], mlp_w2[l], mlp_b2[l][None],
                  bn_gamma[l][None], bn_beta[l][None])

    final = pl.pallas_call(
        _final_body,
        out_shape=(jax.ShapeDtypeStruct((NGRP, feat_w.shape[1]), jnp.float32),
                   jax.ShapeDtypeStruct((NGRP, out_w2.shape[1]), jnp.float32)))
    hf, o = final(batch[None], h, feat_w, feat_b[None], out_w1, out_b1[None],
                  out_w2, out_b2[None])
    return hf, o
